# Initial kernel scaffold; baseline (speedup 1.0000x reference)
import functools

import jax
import jax.numpy as jnp
from jax import lax
from jax.experimental import pallas as pl
from jax.experimental.pallas import tpu as pltpu
from jax.experimental.pallas import tpu_sc as plsc

N = 10000
E = 320000
D_IN = 128
H = 256
HALF = H // 2

NC = 2
NS = 16
EPT = E // NS
K = 80
NCHUNK = EPT // K
RPT = N // NS
RCHUNK = 125
NRC = RPT // RCHUNK

_mesh = plsc.VectorSubcoreMesh(core_axis_name="c", subcore_axis_name="s")


@functools.partial(
    pl.kernel,
    out_type=jax.ShapeDtypeStruct((NC, N, HALF), jnp.float32),
    mesh=_mesh,
    scratch_types=[
        pltpu.VMEM((NCHUNK, K), jnp.int32),
        pltpu.VMEM((NCHUNK, K), jnp.int32),
        pltpu.VMEM((K, HALF), jnp.float32),
        pltpu.VMEM((RCHUNK, HALF), jnp.float32),
        pltpu.VMEM_SHARED((N, HALF), jnp.float32),
    ],
)
def _sc_aggregate(h3, ei, out, src_v, dst_v, rows_v, stage_v, agg_sh):
    c = lax.axis_index("c")
    s = lax.axis_index("s")

    pltpu.sync_copy(ei.at[0].at[s], src_v)
    pltpu.sync_copy(ei.at[1].at[s], dst_v)

    def init_body(j, _):
        base = s * RPT + j * RCHUNK
        pltpu.sync_copy(h3.at[c].at[pl.ds(base, RCHUNK)], stage_v)
        pltpu.sync_copy(stage_v, agg_sh.at[pl.ds(base, RCHUNK)])
        return 0

    lax.fori_loop(0, NRC, init_body, 0)
    plsc.subcore_barrier()

    def edge_body(j, _):
        pltpu.sync_copy(h3.at[c].at[src_v.at[j]], rows_v)
        pltpu.sync_copy(rows_v, agg_sh.at[dst_v.at[j]], add=True)
        return 0

    lax.fori_loop(0, NCHUNK, edge_body, 0)
    plsc.subcore_barrier()

    def out_body(j, _):
        base = s * RPT + j * RCHUNK
        pltpu.sync_copy(agg_sh.at[pl.ds(base, RCHUNK)], stage_v)
        pltpu.sync_copy(stage_v, out.at[c].at[pl.ds(base, RCHUNK)])
        return 0

    lax.fori_loop(0, NRC, out_body, 0)


_BN = 1000


def _encoder_body(x_ref, w_ref, b_ref, out_ref):
    h = jnp.dot(x_ref[...], w_ref[...], preferred_element_type=jnp.float32)
    h = jnp.maximum(h + b_ref[...], 0.0)
    out_ref[0] = h[:, :HALF]
    out_ref[1] = h[:, HALF:]


_encoder = pl.pallas_call(
    _encoder_body,
    grid=(N // _BN,),
    in_specs=[
        pl.BlockSpec((_BN, D_IN), lambda i: (i, 0)),
        pl.BlockSpec((D_IN, H), lambda i: (0, 0)),
        pl.BlockSpec((1, H), lambda i: (0, 0)),
    ],
    out_specs=pl.BlockSpec((NC, _BN, HALF), lambda i: (0, i, 0)),
    out_shape=jax.ShapeDtypeStruct((NC, N, HALF), jnp.float32),
)


def _mlp_body(split_out, z_ref, w1_ref, b1_ref, w2_ref, b2_ref, out_ref):
    w1 = w1_ref[...]
    t = jnp.dot(z_ref[0], w1[:HALF, :], preferred_element_type=jnp.float32)
    t += jnp.dot(z_ref[1], w1[HALF:, :], preferred_element_type=jnp.float32)
    t = jnp.maximum(t + b1_ref[...], 0.0)
    y = jnp.dot(t, w2_ref[...], preferred_element_type=jnp.float32)
    y = jnp.maximum(y + b2_ref[...], 0.0)
    if split_out:
        out_ref[0] = y[:, :HALF]
        out_ref[1] = y[:, HALF:]
    else:
        out_ref[...] = y


def _make_mlp(split_out):
    if split_out:
        out_specs = pl.BlockSpec((NC, _BN, HALF), lambda i: (0, i, 0))
        out_shape = jax.ShapeDtypeStruct((NC, N, HALF), jnp.float32)
    else:
        out_specs = pl.BlockSpec((_BN, H), lambda i: (i, 0))
        out_shape = jax.ShapeDtypeStruct((N, H), jnp.float32)
    return pl.pallas_call(
        functools.partial(_mlp_body, split_out),
        grid=(N // _BN,),
        in_specs=[
            pl.BlockSpec((NC, _BN, HALF), lambda i: (0, i, 0)),
            pl.BlockSpec((H, H), lambda i: (0, 0)),
            pl.BlockSpec((1, H), lambda i: (0, 0)),
            pl.BlockSpec((H, H), lambda i: (0, 0)),
            pl.BlockSpec((1, H), lambda i: (0, 0)),
        ],
        out_specs=out_specs,
        out_shape=out_shape,
    )


_mlp_split = _make_mlp(True)
_mlp_full = _make_mlp(False)


def kernel(x, edge_index, enc_W, enc_b, W1_0, b1_0, W2_0, b2_0,
           W1_1, b1_1, W2_1, b2_1, W1_2, b1_2, W2_2, b2_2):
    ei = edge_index.reshape(2, NS, NCHUNK, K)
    h3 = _encoder(x, enc_W, enc_b.reshape(1, H))
    layers = [(W1_0, b1_0, W2_0, b2_0),
              (W1_1, b1_1, W2_1, b2_1),
              (W1_2, b1_2, W2_2, b2_2)]
    for l, (W1, b1, W2, b2) in enumerate(layers):
        zsum = _sc_aggregate(h3, ei)
        if l < 2:
            h3 = _mlp_split(zsum, W1, b1.reshape(1, H), W2, b2.reshape(1, H))
        else:
            out = _mlp_full(zsum, W1, b1.reshape(1, H), W2, b2.reshape(1, H))
    return out


# speedup vs baseline: 2.8600x; 2.8600x over previous
import functools

import jax
import jax.numpy as jnp
from jax import lax
from jax.experimental import pallas as pl
from jax.experimental.pallas import tpu as pltpu
from jax.experimental.pallas import tpu_sc as plsc

N = 10000
NP = 10240
E = 320000
D_IN = 128
H = 256
NQ = 2
FQ = H // NQ

NC = 2
NS = 16
EPT = E // NS
K = 80
NCHUNK = EPT // K

W = 5016
AGGR = 5024
TRASH = W
RT = 312
RT_LAST_I = AGGR - 15 * RT
RT_LAST_O = W - 15 * RT

_mesh = plsc.VectorSubcoreMesh(
    core_axis_name="c", subcore_axis_name="s", num_cores=NC, num_subcores=NS)


@functools.partial(
    pl.kernel,
    out_type=jax.ShapeDtypeStruct((NQ, NP, FQ), jnp.float32),
    mesh=_mesh,
    scratch_types=[
        pltpu.VMEM((NCHUNK, K), jnp.int32),
        pltpu.VMEM((NCHUNK, K), jnp.int32),
        pltpu.VMEM((K, FQ), jnp.float32),
        pltpu.VMEM_SHARED((AGGR, FQ), jnp.float32),
    ],
)
def _sc_aggregate(h2, eis, eid, out, src_v, dst_v, rows_v, agg_sh):
    c = lax.axis_index("c")
    s = lax.axis_index("s")

    pltpu.sync_copy(eis.at[s], src_v)

    for r in range(2):
        base = r * W

        pltpu.sync_copy(eid.at[r].at[s], dst_v)

        @pl.when(s < 15)
        def _():
            pltpu.sync_copy(h2.at[c].at[pl.ds(base + s * RT, RT)],
                            agg_sh.at[pl.ds(s * RT, RT)])

        @pl.when(s == 15)
        def _():
            pltpu.sync_copy(h2.at[c].at[pl.ds(base + 15 * RT, RT_LAST_I)],
                            agg_sh.at[pl.ds(15 * RT, RT_LAST_I)])

        plsc.subcore_barrier()

        def edge_body(j, _):
            pltpu.sync_copy(h2.at[c].at[src_v.at[j]], rows_v)
            pltpu.sync_copy(rows_v, agg_sh.at[dst_v.at[j]], add=True)
            return 0

        lax.fori_loop(0, NCHUNK, edge_body, 0)
        plsc.subcore_barrier()

        @pl.when(s < 15)
        def _():
            pltpu.sync_copy(agg_sh.at[pl.ds(s * RT, RT)],
                            out.at[c].at[pl.ds(base + s * RT, RT)])

        @pl.when(s == 15)
        def _():
            pltpu.sync_copy(agg_sh.at[pl.ds(15 * RT, RT_LAST_O)],
                            out.at[c].at[pl.ds(base + 15 * RT, RT_LAST_O)])

        plsc.subcore_barrier()


_BN = 1024


def _splitq(y):
    return [y[:, i * FQ:(i + 1) * FQ] for i in range(NQ)]


def _encoder_body(x_ref, w_ref, b_ref, out_ref):
    h = jnp.dot(x_ref[...], w_ref[...], preferred_element_type=jnp.float32)
    h = jnp.maximum(h + b_ref[...], 0.0)
    for i, part in enumerate(_splitq(h)):
        out_ref[i] = part


_encoder = pl.pallas_call(
    _encoder_body,
    grid=(NP // _BN,),
    in_specs=[
        pl.BlockSpec((_BN, D_IN), lambda i: (i, 0)),
        pl.BlockSpec((D_IN, H), lambda i: (0, 0)),
        pl.BlockSpec((1, H), lambda i: (0, 0)),
    ],
    out_specs=pl.BlockSpec((NQ, _BN, FQ), lambda i: (0, i, 0)),
    out_shape=jax.ShapeDtypeStruct((NQ, NP, FQ), jnp.float32),
)


def _mlp_body(split_out, z_ref, w1_ref, b1_ref, w2_ref, b2_ref, out_ref):
    w1 = w1_ref[...]
    t = jnp.dot(z_ref[0], w1[:FQ, :], preferred_element_type=jnp.float32)
    for i in range(1, NQ):
        t += jnp.dot(z_ref[i], w1[i * FQ:(i + 1) * FQ, :],
                     preferred_element_type=jnp.float32)
    t = jnp.maximum(t + b1_ref[...], 0.0)
    y = jnp.dot(t, w2_ref[...], preferred_element_type=jnp.float32)
    y = jnp.maximum(y + b2_ref[...], 0.0)
    if split_out:
        for i, part in enumerate(_splitq(y)):
            out_ref[i] = part
    else:
        out_ref[...] = y


def _make_mlp(split_out):
    if split_out:
        out_specs = pl.BlockSpec((NQ, _BN, FQ), lambda i: (0, i, 0))
        out_shape = jax.ShapeDtypeStruct((NQ, NP, FQ), jnp.float32)
    else:
        out_specs = pl.BlockSpec((_BN, H), lambda i: (i, 0))
        out_shape = jax.ShapeDtypeStruct((NP, H), jnp.float32)
    return pl.pallas_call(
        functools.partial(_mlp_body, split_out),
        grid=(NP // _BN,),
        in_specs=[
            pl.BlockSpec((NQ, _BN, FQ), lambda i: (0, i, 0)),
            pl.BlockSpec((H, H), lambda i: (0, 0)),
            pl.BlockSpec((1, H), lambda i: (0, 0)),
            pl.BlockSpec((H, H), lambda i: (0, 0)),
            pl.BlockSpec((1, H), lambda i: (0, 0)),
        ],
        out_specs=out_specs,
        out_shape=out_shape,
    )


_mlp_split = _make_mlp(True)
_mlp_full = _make_mlp(False)


def kernel(x, edge_index, enc_W, enc_b, W1_0, b1_0, W2_0, b2_0,
           W1_1, b1_1, W2_1, b2_1, W1_2, b1_2, W2_2, b2_2):
    src = edge_index[0]
    dst = edge_index[1]
    eis = src.reshape(NS, NCHUNK, K)
    d0 = jnp.where(dst < W, dst, TRASH)
    d1 = jnp.where(dst >= W, dst - W, TRASH)
    eid = jnp.stack([d0, d1]).reshape(2, NS, NCHUNK, K)
    xp = jnp.pad(x, ((0, NP - N), (0, 0)))
    h2 = _encoder(xp, enc_W, enc_b.reshape(1, H))
    layers = [(W1_0, b1_0, W2_0, b2_0),
              (W1_1, b1_1, W2_1, b2_1),
              (W1_2, b1_2, W2_2, b2_2)]
    for l, (W1, b1, W2, b2) in enumerate(layers):
        zsum = _sc_aggregate(h2, eis, eid)
        if l < 2:
            h2 = _mlp_split(zsum, W1, b1.reshape(1, H), W2, b2.reshape(1, H))
        else:
            out = _mlp_full(zsum, W1, b1.reshape(1, H), W2, b2.reshape(1, H))
    return out[:N]


# speedup vs baseline: 3.7959x; 1.3272x over previous
import functools

import jax
import jax.numpy as jnp
from jax import lax
from jax.experimental import pallas as pl
from jax.experimental.pallas import tpu as pltpu
from jax.experimental.pallas import tpu_sc as plsc

N = 10000
NP = 10240
E = 320000
D_IN = 128
H = 256
NQ = 2
FQ = H // NQ

NC = 2
NS = 16
EPT = E // NS
K = 80
NCHUNK = EPT // K

W = 5016
AGGR = 5024
TRASH = W
RT = 312
RT_LAST_I = AGGR - 15 * RT
RT_LAST_O = W - 15 * RT

_mesh = plsc.VectorSubcoreMesh(
    core_axis_name="c", subcore_axis_name="s", num_cores=NC, num_subcores=NS)


@functools.partial(
    pl.kernel,
    out_type=jax.ShapeDtypeStruct((NQ, NP, FQ), jnp.float32),
    mesh=_mesh,
    scratch_types=[
        pltpu.VMEM((NCHUNK, K), jnp.int32),
        pltpu.VMEM((NCHUNK, K), jnp.int32),
        pltpu.VMEM((2, K, FQ), jnp.float32),
        pltpu.VMEM_SHARED((AGGR, FQ), jnp.float32),
        pltpu.SemaphoreType.DMA,
    ],
)
def _sc_aggregate(h2, eis, eid, out, src_v, dst_v, rows_v, agg_sh, gsem):
    c = lax.axis_index("c")
    s = lax.axis_index("s")

    pltpu.sync_copy(eis.at[s], src_v)

    for r in range(2):
        base = r * W

        pltpu.sync_copy(eid.at[r].at[s], dst_v)

        @pl.when(s < 15)
        def _():
            pltpu.sync_copy(h2.at[c].at[pl.ds(base + s * RT, RT)],
                            agg_sh.at[pl.ds(s * RT, RT)])

        @pl.when(s == 15)
        def _():
            pltpu.sync_copy(h2.at[c].at[pl.ds(base + 15 * RT, RT_LAST_I)],
                            agg_sh.at[pl.ds(15 * RT, RT_LAST_I)])

        plsc.subcore_barrier()

        pltpu.async_copy(h2.at[c].at[src_v.at[0]], rows_v.at[0], gsem)

        def group_body(g, _):
            for b in range(2):
                j = g * 2 + b
                pltpu.make_async_copy(
                    h2.at[c].at[src_v.at[j]], rows_v.at[b], gsem).wait()

                @pl.when(j + 1 < NCHUNK)
                def _():
                    pltpu.async_copy(
                        h2.at[c].at[src_v.at[j + 1]], rows_v.at[1 - b], gsem)

                pltpu.sync_copy(rows_v.at[b], agg_sh.at[dst_v.at[j]], add=True)
            return 0

        lax.fori_loop(0, NCHUNK // 2, group_body, 0)
        plsc.subcore_barrier()

        @pl.when(s < 15)
        def _():
            pltpu.sync_copy(agg_sh.at[pl.ds(s * RT, RT)],
                            out.at[c].at[pl.ds(base + s * RT, RT)])

        @pl.when(s == 15)
        def _():
            pltpu.sync_copy(agg_sh.at[pl.ds(15 * RT, RT_LAST_O)],
                            out.at[c].at[pl.ds(base + 15 * RT, RT_LAST_O)])

        plsc.subcore_barrier()


_BN = 1024


def _splitq(y):
    return [y[:, i * FQ:(i + 1) * FQ] for i in range(NQ)]


def _encoder_body(x_ref, w_ref, b_ref, out_ref):
    h = jnp.dot(x_ref[...], w_ref[...], preferred_element_type=jnp.float32)
    h = jnp.maximum(h + b_ref[...], 0.0)
    for i, part in enumerate(_splitq(h)):
        out_ref[i] = part


_encoder = pl.pallas_call(
    _encoder_body,
    grid=(NP // _BN,),
    in_specs=[
        pl.BlockSpec((_BN, D_IN), lambda i: (i, 0)),
        pl.BlockSpec((D_IN, H), lambda i: (0, 0)),
        pl.BlockSpec((1, H), lambda i: (0, 0)),
    ],
    out_specs=pl.BlockSpec((NQ, _BN, FQ), lambda i: (0, i, 0)),
    out_shape=jax.ShapeDtypeStruct((NQ, NP, FQ), jnp.float32),
)


def _mlp_body(split_out, z_ref, w1_ref, b1_ref, w2_ref, b2_ref, out_ref):
    w1 = w1_ref[...]
    t = jnp.dot(z_ref[0], w1[:FQ, :], preferred_element_type=jnp.float32)
    for i in range(1, NQ):
        t += jnp.dot(z_ref[i], w1[i * FQ:(i + 1) * FQ, :],
                     preferred_element_type=jnp.float32)
    t = jnp.maximum(t + b1_ref[...], 0.0)
    y = jnp.dot(t, w2_ref[...], preferred_element_type=jnp.float32)
    y = jnp.maximum(y + b2_ref[...], 0.0)
    if split_out:
        for i, part in enumerate(_splitq(y)):
            out_ref[i] = part
    else:
        out_ref[...] = y


def _make_mlp(split_out):
    if split_out:
        out_specs = pl.BlockSpec((NQ, _BN, FQ), lambda i: (0, i, 0))
        out_shape = jax.ShapeDtypeStruct((NQ, NP, FQ), jnp.float32)
    else:
        out_specs = pl.BlockSpec((_BN, H), lambda i: (i, 0))
        out_shape = jax.ShapeDtypeStruct((NP, H), jnp.float32)
    return pl.pallas_call(
        functools.partial(_mlp_body, split_out),
        grid=(NP // _BN,),
        in_specs=[
            pl.BlockSpec((NQ, _BN, FQ), lambda i: (0, i, 0)),
            pl.BlockSpec((H, H), lambda i: (0, 0)),
            pl.BlockSpec((1, H), lambda i: (0, 0)),
            pl.BlockSpec((H, H), lambda i: (0, 0)),
            pl.BlockSpec((1, H), lambda i: (0, 0)),
        ],
        out_specs=out_specs,
        out_shape=out_shape,
    )


_mlp_split = _make_mlp(True)
_mlp_full = _make_mlp(False)


def kernel(x, edge_index, enc_W, enc_b, W1_0, b1_0, W2_0, b2_0,
           W1_1, b1_1, W2_1, b2_1, W1_2, b1_2, W2_2, b2_2):
    src = edge_index[0]
    dst = edge_index[1]
    eis = src.reshape(NS, NCHUNK, K)
    d0 = jnp.where(dst < W, dst, TRASH)
    d1 = jnp.where(dst >= W, dst - W, TRASH)
    eid = jnp.stack([d0, d1]).reshape(2, NS, NCHUNK, K)
    xp = jnp.pad(x, ((0, NP - N), (0, 0)))
    h2 = _encoder(xp, enc_W, enc_b.reshape(1, H))
    layers = [(W1_0, b1_0, W2_0, b2_0),
              (W1_1, b1_1, W2_1, b2_1),
              (W1_2, b1_2, W2_2, b2_2)]
    for l, (W1, b1, W2, b2) in enumerate(layers):
        zsum = _sc_aggregate(h2, eis, eid)
        if l < 2:
            h2 = _mlp_split(zsum, W1, b1.reshape(1, H), W2, b2.reshape(1, H))
        else:
            out = _mlp_full(zsum, W1, b1.reshape(1, H), W2, b2.reshape(1, H))
    return out[:N]
